# baseline (speedup 1.0000x reference)
import functools

import jax
import jax.numpy as jnp
from jax import lax
from jax.experimental import pallas as pl
from jax.experimental.pallas import tpu as pltpu
from jax.experimental.pallas import tpu_sc as plsc

B, INC, INN, OUTC, OUTN, D = 4, 128, 100000, 128, 25000, 16
BC = B * INC
NC_SC, NS_SC = 2, 16
NW = NC_SC * NS_SC
OUTN_PAD = 25600
NPW = OUTN_PAD // NW
G = 4
GI = G * D
NCHUNK = NPW // G
IPW = NPW * D


NB1 = 1024


def _mulT_body(x_ref, w_ref, out_ref):
    b = pl.program_id(1)
    h = x_ref[0] * w_ref[...]
    eye = jnp.eye(INC, dtype=jnp.float32)
    t = lax.dot_general(h, eye, (((0,), (0,)), ((), ())),
                        preferred_element_type=jnp.float32)
    out_ref[:, pl.ds(b * INC, INC)] = t


def _mulT(x, w):
    grid = (pl.cdiv(INN, NB1), B)
    return pl.pallas_call(
        _mulT_body,
        grid=grid,
        in_specs=[
            pl.BlockSpec((1, INC, NB1), lambda n, b: (b, 0, n)),
            pl.BlockSpec((INC, NB1), lambda n, b: (0, n)),
        ],
        out_specs=pl.BlockSpec((NB1, BC), lambda n, b: (n, 0)),
        out_shape=jax.ShapeDtypeStruct((INN, BC), jnp.float32),
    )(x, w)


def _sc_body(ht_hbm, adj_hbm, red_hbm,
             idx_all, rows0, rows1, red0, red1, sg0, sg1, so0, so1):
    cid = lax.axis_index("c")
    sid = lax.axis_index("s")
    wid = sid * NC_SC + cid
    nbase = wid * NPW

    pltpu.sync_copy(adj_hbm.at[pl.ds(wid * IPW, IPW)], idx_all)

    def gather_start(chunk, rows, sem):
        idx = idx_all.at[pl.ds(chunk * GI, GI)]
        pltpu.async_copy(ht_hbm.at[idx], rows, sem)

    def gather_wait(rows, sem):
        pltpu.make_async_copy(ht_hbm.at[idx_all.at[pl.ds(0, GI)]], rows,
                              sem).wait()

    def out_start(chunk, red, sem):
        pltpu.async_copy(red, red_hbm.at[pl.ds(nbase + chunk * G, G)], sem)

    def out_wait(red, sem):
        pltpu.make_async_copy(red, red_hbm.at[pl.ds(nbase, G)], sem).wait()

    def reduce_chunk(rows, red):
        def cbody(c, _):
            off = c * 16
            for g in range(G):
                v = [rows[g * D + d, pl.ds(off, 16)] for d in range(D)]
                while len(v) > 1:
                    v = [jnp.maximum(v[k], v[k + 1])
                         for k in range(0, len(v) - 1, 2)] + v[len(v) & ~1:]
                red[g, pl.ds(off, 16)] = v[0]
            return 0

        lax.fori_loop(0, BC // 16, cbody, 0)

    gather_start(0, rows0, sg0)
    gather_start(1, rows1, sg1)

    def pair(p, _):
        c0 = 2 * p

        gather_wait(rows0, sg0)

        @pl.when(p > 0)
        def _():
            out_wait(red0, so0)

        reduce_chunk(rows0, red0)
        out_start(c0, red0, so0)

        @pl.when(c0 + 2 < NCHUNK)
        def _():
            gather_start(c0 + 2, rows0, sg0)

        gather_wait(rows1, sg1)

        @pl.when(p > 0)
        def _():
            out_wait(red1, so1)

        reduce_chunk(rows1, red1)
        out_start(c0 + 1, red1, so1)

        @pl.when(c0 + 3 < NCHUNK)
        def _():
            gather_start(c0 + 3, rows1, sg1)

        return 0

    lax.fori_loop(0, NCHUNK // 2, pair, 0)
    out_wait(red0, so0)
    out_wait(red1, so1)


_sc_call = functools.partial(
    pl.kernel,
    out_type=jax.ShapeDtypeStruct((OUTN_PAD, BC), jnp.float32),
    mesh=plsc.VectorSubcoreMesh(core_axis_name="c", subcore_axis_name="s"),
    scratch_types=[
        pltpu.VMEM((IPW,), jnp.int32),
        pltpu.VMEM((GI, BC), jnp.float32),
        pltpu.VMEM((GI, BC), jnp.float32),
        pltpu.VMEM((G, BC), jnp.float32),
        pltpu.VMEM((G, BC), jnp.float32),
        pltpu.SemaphoreType.DMA,
        pltpu.SemaphoreType.DMA,
        pltpu.SemaphoreType.DMA,
        pltpu.SemaphoreType.DMA,
    ],
    compiler_params=pltpu.CompilerParams(needs_layout_passes=False),
)(_sc_body)


NB3 = 1024


def _dot_body(red_ref, ft_ref, bias_ref, out_ref):
    b = pl.program_id(1)
    red = red_ref[:, pl.ds(b * INC, INC)]
    t = lax.dot_general(ft_ref[...], red, (((0,), (1,)), ((), ())),
                        preferred_element_type=jnp.float32)
    out_ref[0] = t + bias_ref[0]


def _dot(red, ft, bias):
    grid = (pl.cdiv(OUTN, NB3), B)
    return pl.pallas_call(
        _dot_body,
        grid=grid,
        in_specs=[
            pl.BlockSpec((NB3, BC), lambda n, b: (n, 0)),
            pl.BlockSpec((INC, OUTC), lambda n, b: (0, 0)),
            pl.BlockSpec((1, OUTC, NB3), lambda n, b: (0, 0, n)),
        ],
        out_specs=pl.BlockSpec((1, OUTC, NB3), lambda n, b: (b, 0, n)),
        out_shape=jax.ShapeDtypeStruct((B, OUTC, OUTN), jnp.float32),
    )(red, ft, bias)


def kernel(x, adj, nf_weight, ft_weight, bias):
    ht = _mulT(x, nf_weight)
    adj_pad = jnp.zeros((OUTN_PAD, D), jnp.int32).at[:OUTN].set(adj)
    red = _sc_call(ht, adj_pad.reshape(-1))
    out = _dot(red, ft_weight, bias)
    return out


# speedup vs baseline: 1.0601x; 1.0601x over previous
import functools

import jax
import jax.numpy as jnp
from jax import lax
from jax.experimental import pallas as pl
from jax.experimental.pallas import tpu as pltpu
from jax.experimental.pallas import tpu_sc as plsc

B, INC, INN, OUTC, OUTN, D = 4, 128, 100000, 128, 25000, 16
BC = B * INC
NC_SC, NS_SC = 2, 16
NW = NC_SC * NS_SC
OUTN_PAD = 25600
G = 4
GI = G * D
NF = 1024
NS = 576
FAST_CID = 0


NB1 = 1024


def _mulT_body(x_ref, w_ref, out_ref):
    b = pl.program_id(1)
    h = x_ref[0] * w_ref[...]
    eye = jnp.eye(INC, dtype=jnp.float32)
    t = lax.dot_general(h, eye, (((0,), (0,)), ((), ())),
                        preferred_element_type=jnp.float32)
    out_ref[:, pl.ds(b * INC, INC)] = t


def _mulT(x, w):
    grid = (pl.cdiv(INN, NB1), B)
    return pl.pallas_call(
        _mulT_body,
        grid=grid,
        in_specs=[
            pl.BlockSpec((1, INC, NB1), lambda n, b: (b, 0, n)),
            pl.BlockSpec((INC, NB1), lambda n, b: (0, n)),
        ],
        out_specs=pl.BlockSpec((NB1, BC), lambda n, b: (n, 0)),
        out_shape=jax.ShapeDtypeStruct((INN, BC), jnp.float32),
    )(x, w)


def _sc_body(ht_hbm, adj_hbm, red_hbm,
             idx_all, idxb0, idxb1, rows0, rows1, red0, red1,
             sg0, sg1, so0, so1):
    cid = lax.axis_index("c")
    sid = lax.axis_index("s")
    fast = cid == FAST_CID
    npw = jnp.where(fast, NF, NS)
    nbase = jnp.where(fast, sid * NF, 16 * NF + sid * NS)
    nchunk = npw // G

    @pl.when(fast)
    def _():
        pltpu.sync_copy(adj_hbm.at[pl.ds(nbase * D, NF * D)], idx_all)

    @pl.when(jnp.logical_not(fast))
    def _():
        pltpu.sync_copy(adj_hbm.at[pl.ds(nbase * D, NS * D)],
                        idx_all.at[pl.ds(0, NS * D)])

    def idx_prep(chunk, idxb):
        for k in range(GI // 16):
            idxb[pl.ds(k * 16, 16)] = idx_all[pl.ds(chunk * GI + k * 16, 16)]

    def gather_start(rows, idxb, sem):
        pltpu.async_copy(ht_hbm.at[idxb], rows, sem)

    def gather_wait(rows, idxb, sem):
        pltpu.make_async_copy(ht_hbm.at[idxb], rows, sem).wait()

    def out_start(chunk, red, sem):
        pltpu.async_copy(red, red_hbm.at[pl.ds(nbase + chunk * G, G)], sem)

    def out_wait(red, sem):
        pltpu.make_async_copy(red, red_hbm.at[pl.ds(nbase, G)], sem).wait()

    def reduce_chunk(rows, red):
        def cbody(c, _):
            off = c * 16
            for g in range(G):
                v = [rows[g * D + d, pl.ds(off, 16)] for d in range(D)]
                while len(v) > 1:
                    v = [jnp.maximum(v[k], v[k + 1])
                         for k in range(0, len(v) - 1, 2)] + v[len(v) & ~1:]
                red[g, pl.ds(off, 16)] = v[0]
            return 0

        lax.fori_loop(0, BC // 16, cbody, 0)

    idx_prep(0, idxb0)
    gather_start(rows0, idxb0, sg0)
    idx_prep(1, idxb1)
    gather_start(rows1, idxb1, sg1)

    def pair(p, _):
        c0 = 2 * p

        gather_wait(rows0, idxb0, sg0)

        @pl.when(p > 0)
        def _():
            out_wait(red0, so0)

        reduce_chunk(rows0, red0)
        out_start(c0, red0, so0)

        @pl.when(c0 + 2 < nchunk)
        def _():
            idx_prep(c0 + 2, idxb0)
            gather_start(rows0, idxb0, sg0)

        gather_wait(rows1, idxb1, sg1)

        @pl.when(p > 0)
        def _():
            out_wait(red1, so1)

        reduce_chunk(rows1, red1)
        out_start(c0 + 1, red1, so1)

        @pl.when(c0 + 3 < nchunk)
        def _():
            idx_prep(c0 + 3, idxb1)
            gather_start(rows1, idxb1, sg1)

        return 0

    lax.fori_loop(0, nchunk // 2, pair, 0)
    out_wait(red0, so0)
    out_wait(red1, so1)


_sc_call = functools.partial(
    pl.kernel,
    out_type=jax.ShapeDtypeStruct((OUTN_PAD, BC), jnp.float32),
    mesh=plsc.VectorSubcoreMesh(core_axis_name="c", subcore_axis_name="s"),
    scratch_types=[
        pltpu.VMEM((NF * D,), jnp.int32),
        pltpu.VMEM((GI,), jnp.int32),
        pltpu.VMEM((GI,), jnp.int32),
        pltpu.VMEM((GI, BC), jnp.float32),
        pltpu.VMEM((GI, BC), jnp.float32),
        pltpu.VMEM((G, BC), jnp.float32),
        pltpu.VMEM((G, BC), jnp.float32),
        pltpu.SemaphoreType.DMA,
        pltpu.SemaphoreType.DMA,
        pltpu.SemaphoreType.DMA,
        pltpu.SemaphoreType.DMA,
    ],
    compiler_params=pltpu.CompilerParams(needs_layout_passes=False),
)(_sc_body)


NB3 = 1024


def _dot_body(red_ref, ft_ref, bias_ref, out_ref):
    b = pl.program_id(1)
    red = red_ref[:, pl.ds(b * INC, INC)]
    t = lax.dot_general(ft_ref[...], red, (((0,), (1,)), ((), ())),
                        preferred_element_type=jnp.float32)
    out_ref[0] = t + bias_ref[0]


def _dot(red, ft, bias):
    grid = (pl.cdiv(OUTN, NB3), B)
    return pl.pallas_call(
        _dot_body,
        grid=grid,
        in_specs=[
            pl.BlockSpec((NB3, BC), lambda n, b: (n, 0)),
            pl.BlockSpec((INC, OUTC), lambda n, b: (0, 0)),
            pl.BlockSpec((1, OUTC, NB3), lambda n, b: (0, 0, n)),
        ],
        out_specs=pl.BlockSpec((1, OUTC, NB3), lambda n, b: (b, 0, n)),
        out_shape=jax.ShapeDtypeStruct((B, OUTC, OUTN), jnp.float32),
    )(red, ft, bias)


def kernel(x, adj, nf_weight, ft_weight, bias):
    ht = _mulT(x, nf_weight)
    adj_pad = jnp.zeros((OUTN_PAD, D), jnp.int32).at[:OUTN].set(adj)
    red = _sc_call(ht, adj_pad.reshape(-1))
    out = _dot(red, ft_weight, bias)
    return out
